# baseline (speedup 1.0000x reference)
import functools

import jax
import jax.numpy as jnp
from jax import lax
from jax.experimental import pallas as pl
from jax.experimental.pallas import tpu as pltpu
from jax.experimental.pallas import tpu_sc as plsc

NUM_CORES = 2
NUM_SUBCORES = 16
NUM_WORKERS = NUM_CORES * NUM_SUBCORES

CHUNK = 1024


def _gather_rows(n_total: int, dim: int):
  n_per_w = n_total // NUM_WORKERS
  n_chunks = n_per_w // CHUNK
  mesh = plsc.VectorSubcoreMesh(
      core_axis_name="c", subcore_axis_name="s",
      num_cores=NUM_CORES, num_subcores=NUM_SUBCORES)

  @functools.partial(
      pl.kernel,
      mesh=mesh,
      out_type=jax.ShapeDtypeStruct((n_total, dim), jnp.float32),
      scratch_types=[
          pltpu.VMEM((CHUNK,), jnp.int32),
          pltpu.VMEM((CHUNK, dim), jnp.float32),
          pltpu.SemaphoreType.DMA,
      ],
      compiler_params=pltpu.CompilerParams(use_tc_tiling_on_sc=False),
  )
  def grab(idx_hbm, table_hbm, out_hbm, idx_v, rows_v, sem):
    wid = lax.axis_index("s") * NUM_CORES + lax.axis_index("c")
    base = wid * n_per_w

    @pl.loop(0, n_chunks)
    def _step(i):
      off = base + i * CHUNK
      pltpu.sync_copy(idx_hbm.at[pl.ds(off, CHUNK)], idx_v)
      pltpu.async_copy(table_hbm.at[idx_v], rows_v, sem).wait()
      pltpu.sync_copy(rows_v, out_hbm.at[pl.ds(off, CHUNK)])

  return grab


def kernel(items, weight):
  batch, hist = items.shape
  vocab, dim = weight.shape
  n_total = batch * hist
  flat_idx = items.reshape(n_total).astype(jnp.int32)
  out = _gather_rows(n_total, dim)(flat_idx, weight)
  return out.reshape(batch, hist, dim)


# speedup vs baseline: 1.0164x; 1.0164x over previous
import functools

import jax
import jax.numpy as jnp
from jax import lax
from jax.experimental import pallas as pl
from jax.experimental.pallas import tpu as pltpu
from jax.experimental.pallas import tpu_sc as plsc

NUM_CORES = 2
NUM_SUBCORES = 16
NUM_WORKERS = NUM_CORES * NUM_SUBCORES

CHUNK = 640
NBUF = 2


def _gather_rows(n_total: int, dim: int):
  n_per_w = n_total // NUM_WORKERS
  n_chunks = n_per_w // CHUNK
  assert n_chunks % NBUF == 0
  mesh = plsc.VectorSubcoreMesh(
      core_axis_name="c", subcore_axis_name="s",
      num_cores=NUM_CORES, num_subcores=NUM_SUBCORES)

  @functools.partial(
      pl.kernel,
      mesh=mesh,
      out_type=jax.ShapeDtypeStruct((n_total, dim), jnp.float32),
      scratch_types=[
          pltpu.VMEM((n_per_w,), jnp.int32),
          pltpu.VMEM((NBUF, CHUNK, dim), jnp.float32),
          pltpu.SemaphoreType.DMA((NBUF,)),
          pltpu.SemaphoreType.DMA((NBUF,)),
      ],
      compiler_params=pltpu.CompilerParams(use_tc_tiling_on_sc=False),
  )
  def grab(idx_hbm, table_hbm, out_hbm, idx_v, rows_v, gsem, wsem):
    wid = lax.axis_index("s") * NUM_CORES + lax.axis_index("c")
    base = wid * n_per_w
    pltpu.sync_copy(idx_hbm.at[pl.ds(base, n_per_w)], idx_v)

    for b in range(NBUF):
      pltpu.async_copy(
          table_hbm.at[idx_v.at[pl.ds(b * CHUNK, CHUNK)]],
          rows_v.at[b], gsem.at[b])

    @pl.loop(0, n_chunks, step=NBUF)
    def _step(i):
      for b in range(NBUF):
        cur = i + b
        pltpu.make_async_copy(
            table_hbm.at[idx_v.at[pl.ds(cur * CHUNK, CHUNK)]],
            rows_v.at[b], gsem.at[b]).wait()
        pltpu.async_copy(
            rows_v.at[b], out_hbm.at[pl.ds(base + cur * CHUNK, CHUNK)],
            wsem.at[b])
        nxt = cur + NBUF

        @pl.when(nxt < n_chunks)
        def _():
          pltpu.make_async_copy(
              rows_v.at[b], out_hbm.at[pl.ds(base + cur * CHUNK, CHUNK)],
              wsem.at[b]).wait()
          pltpu.async_copy(
              table_hbm.at[idx_v.at[pl.ds(nxt * CHUNK, CHUNK)]],
              rows_v.at[b], gsem.at[b])

    for b in range(NBUF):
      cur = n_chunks - NBUF + b
      pltpu.make_async_copy(
          rows_v.at[b], out_hbm.at[pl.ds(base + cur * CHUNK, CHUNK)],
          wsem.at[b]).wait()

  return grab


def kernel(items, weight):
  batch, hist = items.shape
  vocab, dim = weight.shape
  n_total = batch * hist
  flat_idx = items.reshape(n_total).astype(jnp.int32)
  out = _gather_rows(n_total, dim)(flat_idx, weight)
  return out.reshape(batch, hist, dim)
